# baseline (speedup 1.0000x reference)
import functools

import jax
import jax.numpy as jnp
from jax import lax
from jax.experimental import pallas as pl
from jax.experimental.pallas import tpu as pltpu
from jax.experimental.pallas import tpu_sc as plsc

_NC = 2
_NS = 16
_NW = _NC * _NS
_L = 16


def _bitrev4(i: int) -> int:
    return ((i & 1) << 3) | ((i & 2) << 1) | ((i & 4) >> 1) | ((i & 8) >> 3)


_GATHER_DNUMS = lax.GatherDimensionNumbers(
    offset_dims=(), collapsed_slice_dims=(0,), start_index_map=(0,))


def _take(v, idx):
    return lax.gather(v, idx[:, None], _GATHER_DNUMS, slice_sizes=(1,),
                      mode=lax.GatherScatterMode.PROMISE_IN_BOUNDS)


def _make_sc_kernel(E: int, D: int, C: int, V: int):
    e_per_w = E // _NW
    n_chunks = e_per_w // C
    assert e_per_w * _NW == E and n_chunks * C == e_per_w
    assert C % _L == 0 and C <= 128 and D % _L == 0

    mesh = plsc.VectorSubcoreMesh(core_axis_name="c", subcore_axis_name="s")

    @functools.partial(
        pl.kernel,
        out_type=jax.ShapeDtypeStruct((E,), jnp.float32),
        mesh=mesh,
        compiler_params=pltpu.CompilerParams(use_tc_tiling_on_sc=False),
        scratch_types=[
            pltpu.VMEM((n_chunks, C), jnp.int32),
            pltpu.VMEM((n_chunks, C), jnp.int32),
            pltpu.VMEM((C, D // 2), jnp.int32),
            pltpu.VMEM((C, D // 2), jnp.int32),
            pltpu.VMEM((C, D // 2), jnp.int32),
            pltpu.VMEM((C, D // 2), jnp.int32),
            pltpu.VMEM((C,), jnp.float32),
            pltpu.VMEM((C,), jnp.float32),
            pltpu.VMEM((C, _L), jnp.float32),
            pltpu.VMEM_SHARED((V, D // 2), jnp.int32),
            pltpu.VMEM_SHARED((V, D // 2), jnp.int32),
            pltpu.SemaphoreType.DMA,
            pltpu.SemaphoreType.DMA,
            pltpu.SemaphoreType.DMA,
        ],
    )
    def sc_kernel(title, label, heads, tails, out, hidx, tidx,
                  hrows0, hrows1, trows0, trows1, obuf0, obuf1, pbuf,
                  stitle, slabel, sem0, sem1, sem_out):
        wid = lax.axis_index("s") * _NC + lax.axis_index("c")
        base = wid * e_per_w

        @pl.when(lax.axis_index("s") == 0)
        def _():
            pltpu.sync_copy(title, stitle)
            pltpu.sync_copy(label, slabel)

        plsc.subcore_barrier()
        hrows = (hrows0, hrows1)
        trows = (trows0, trows1)
        obufs = (obuf0, obuf1)
        sems = (sem0, sem1)
        iota = lax.iota(jnp.int32, _L)
        perms = [iota ^ hw for hw in (8, 4, 2, 1)]
        masks = [(iota & hw) == 0 for hw in (8, 4, 2, 1)]

        pltpu.sync_copy(heads.at[wid], hidx)
        pltpu.sync_copy(tails.at[wid], tidx)

        def fire(cur, slot):
            half = C // 2
            for k in range(2):
                sl = pl.ds(k * half, half)
                pltpu.async_copy(stitle.at[hidx.at[cur, sl]],
                                 hrows[slot].at[sl], sems[slot])
                pltpu.async_copy(slabel.at[tidx.at[cur, sl]],
                                 trows[slot].at[sl], sems[slot])

        def drain(slot):
            pltpu.make_async_copy(stitle.at[hidx.at[0]], hrows[slot],
                                  sems[slot]).wait()
            pltpu.make_async_copy(slabel.at[tidx.at[0]], trows[slot],
                                  sems[slot]).wait()

        def wait_out():
            pltpu.make_async_copy(obuf0, out.at[pl.ds(base, C)],
                                  sem_out).wait()

        def compute(cur, slot):
            hi_mask = jnp.int32(-65536)

            def edge_body(e, ecarry):
                acc_a = None
                acc_b = None
                for g in range(D // (2 * _L)):
                    h = hrows[slot][e, pl.ds(g * _L, _L)]
                    t = trows[slot][e, pl.ds(g * _L, _L)]
                    ha = lax.bitcast_convert_type(h << 16, jnp.float32)
                    hb = lax.bitcast_convert_type(h & hi_mask, jnp.float32)
                    ta = lax.bitcast_convert_type(t << 16, jnp.float32)
                    tb = lax.bitcast_convert_type(t & hi_mask, jnp.float32)
                    acc_a = ha * ta if acc_a is None else acc_a + ha * ta
                    acc_b = hb * tb if acc_b is None else acc_b + hb * tb
                pbuf[e, pl.ds(0, _L)] = acc_a + acc_b
                return ecarry

            lax.fori_loop(0, C, edge_body, 0, unroll=False)

            def group_body(j, gcarry):
                eb = j * _L
                stack = []
                for t in range(_L):
                    v, lvl = pbuf[eb + _bitrev4(t), pl.ds(0, _L)], 0
                    while stack and stack[-1][0] == lvl:
                        _, x = stack.pop()
                        y, perm, mask = v, perms[lvl], masks[lvl]
                        v = jnp.where(mask, x + _take(x, perm),
                                      y + _take(y, perm))
                        lvl += 1
                    stack.append((lvl, v))
                obufs[slot][pl.ds(eb, _L)] = stack[0][1]
                return gcarry

            lax.fori_loop(0, C // _L, group_body, 0, unroll=False)
            pltpu.async_copy(obufs[slot], out.at[pl.ds(base + cur * C, C)],
                             sem_out)

        fire(0, 0)

        def pair_body(i, carry):
            for b in range(2):
                cur = 2 * i + b
                drain(b)

                @pl.when(cur + 1 < n_chunks)
                def _():
                    fire(cur + 1, 1 - b)

                @pl.when(cur >= 2)
                def _():
                    wait_out()

                compute(cur, b)
            return carry

        lax.fori_loop(0, n_chunks // 2, pair_body, 0, unroll=False)
        if n_chunks % 2:
            drain(0)
            wait_out()
            compute(n_chunks - 1, 0)
        for _unused in range(min(n_chunks, 2)):
            wait_out()

    return sc_kernel


def kernel(x_title, x_label, edge_label_index):
    E = edge_label_index.shape[1]
    D = x_title.shape[1]
    C = 80
    e_per_w = E // _NW
    idx = edge_label_index.astype(jnp.int32)
    heads = idx[0].reshape(_NW, e_per_w // C, C)
    tails = idx[1].reshape(_NW, e_per_w // C, C)
    sc = _make_sc_kernel(E, D, C, x_title.shape[0])

    def to_packed(x):
        x16 = x.astype(jnp.bfloat16).reshape(x.shape[0], D // 2, 2)
        return lax.bitcast_convert_type(x16, jnp.int32)

    return sc(to_packed(x_title), to_packed(x_label), heads, tails)


# speedup vs baseline: 1.0909x; 1.0909x over previous
import functools

import jax
import jax.numpy as jnp
from jax import lax
from jax.experimental import pallas as pl
from jax.experimental.pallas import tpu as pltpu
from jax.experimental.pallas import tpu_sc as plsc

_NC = 2
_NS = 16
_NW = _NC * _NS
_L = 16


def _bitrev4(i: int) -> int:
    return ((i & 1) << 3) | ((i & 2) << 1) | ((i & 4) >> 1) | ((i & 8) >> 3)


_GATHER_DNUMS = lax.GatherDimensionNumbers(
    offset_dims=(), collapsed_slice_dims=(0,), start_index_map=(0,))


def _take(v, idx):
    return lax.gather(v, idx[:, None], _GATHER_DNUMS, slice_sizes=(1,),
                      mode=lax.GatherScatterMode.PROMISE_IN_BOUNDS)


def _make_sc_kernel(E: int, D: int, C: int):
    e_per_w = E // _NW
    n_chunks = e_per_w // C
    assert e_per_w * _NW == E and n_chunks * C == e_per_w
    assert C % _L == 0 and C <= 128 and D % _L == 0

    mesh = plsc.VectorSubcoreMesh(core_axis_name="c", subcore_axis_name="s")

    @functools.partial(
        pl.kernel,
        out_type=jax.ShapeDtypeStruct((E,), jnp.float32),
        mesh=mesh,
        scratch_types=[
            pltpu.VMEM((n_chunks, C), jnp.int32),
            pltpu.VMEM((n_chunks, C), jnp.int32),
            pltpu.VMEM((C, D), jnp.float32),
            pltpu.VMEM((C, D), jnp.float32),
            pltpu.VMEM((C, D), jnp.float32),
            pltpu.VMEM((C, D), jnp.float32),
            pltpu.VMEM((e_per_w,), jnp.float32),
            pltpu.VMEM((C, _L), jnp.float32),
            pltpu.SemaphoreType.DMA,
            pltpu.SemaphoreType.DMA,
        ],
    )
    def sc_kernel(title, label, heads, tails, out, hidx, tidx,
                  hrows0, hrows1, trows0, trows1, obuf, pbuf, sem0, sem1):
        wid = lax.axis_index("s") * _NC + lax.axis_index("c")
        hrows = (hrows0, hrows1)
        trows = (trows0, trows1)
        sems = (sem0, sem1)
        iota = lax.iota(jnp.int32, _L)
        perms = [iota ^ hw for hw in (8, 4, 2, 1)]
        masks = [(iota & hw) == 0 for hw in (8, 4, 2, 1)]

        pltpu.sync_copy(heads.at[wid], hidx)
        pltpu.sync_copy(tails.at[wid], tidx)

        def fire(cur, slot):
            pltpu.async_copy(title.at[hidx.at[cur]], hrows[slot], sems[slot])
            pltpu.async_copy(label.at[tidx.at[cur]], trows[slot], sems[slot])

        def drain(slot):
            pltpu.make_async_copy(title.at[hidx.at[0]], hrows[slot],
                                  sems[slot]).wait()
            pltpu.make_async_copy(label.at[tidx.at[0]], trows[slot],
                                  sems[slot]).wait()

        def compute(cur, slot):
            obase = cur * C

            def edge_body(e, ecarry):
                acc_a = (hrows[slot][e, pl.ds(0, _L)]
                         * trows[slot][e, pl.ds(0, _L)])
                acc_b = (hrows[slot][e, pl.ds(_L, _L)]
                         * trows[slot][e, pl.ds(_L, _L)])
                for g in range(2, D // _L, 2):
                    acc_a = acc_a + (hrows[slot][e, pl.ds(g * _L, _L)]
                                     * trows[slot][e, pl.ds(g * _L, _L)])
                    acc_b = acc_b + (hrows[slot][e, pl.ds((g + 1) * _L, _L)]
                                     * trows[slot][e, pl.ds((g + 1) * _L, _L)])
                pbuf[e, pl.ds(0, _L)] = acc_a + acc_b
                return ecarry

            lax.fori_loop(0, C, edge_body, 0, unroll=False)

            def group_body(j, gcarry):
                eb = j * _L
                stack = []
                for t in range(_L):
                    v, lvl = pbuf[eb + _bitrev4(t), pl.ds(0, _L)], 0
                    while stack and stack[-1][0] == lvl:
                        _, x = stack.pop()
                        y, perm, mask = v, perms[lvl], masks[lvl]
                        v = jnp.where(mask, x + _take(x, perm),
                                      y + _take(y, perm))
                        lvl += 1
                    stack.append((lvl, v))
                obuf[pl.ds(obase + eb, _L)] = stack[0][1]
                return gcarry

            lax.fori_loop(0, C // _L, group_body, 0, unroll=False)

        fire(0, 0)

        def pair_body(i, carry):
            for b in range(2):
                cur = 2 * i + b
                drain(b)

                @pl.when(cur + 1 < n_chunks)
                def _():
                    fire(cur + 1, 1 - b)

                compute(cur, b)
            return carry

        lax.fori_loop(0, n_chunks // 2, pair_body, 0, unroll=False)
        if n_chunks % 2:
            drain(0)
            compute(n_chunks - 1, 0)

        pltpu.sync_copy(obuf, out.at[pl.ds(wid * e_per_w, e_per_w)])

    return sc_kernel


def kernel(x_title, x_label, edge_label_index):
    E = edge_label_index.shape[1]
    D = x_title.shape[1]
    C = 80
    e_per_w = E // _NW
    idx = edge_label_index.astype(jnp.int32)
    heads = idx[0].reshape(_NW, e_per_w // C, C)
    tails = idx[1].reshape(_NW, e_per_w // C, C)
    sc = _make_sc_kernel(E, D, C)
    return sc(x_title, x_label, heads, tails)
